# baseline (speedup 1.0000x reference)
import jax
import jax.numpy as jnp
from jax import lax
from jax.experimental import pallas as pl
from jax.experimental.pallas import tpu as pltpu

COLUMN_COUNT = 65536
CELLS_PER_COLUMN = 32
NUM_CELLS = COLUMN_COUNT * CELLS_PER_COLUMN

_ROWS = NUM_CELLS // 128
_IN_COLS = 128 // CELLS_PER_COLUMN
_BR = 2048


def _tc_expand_body(x_ref, o_ref):
    m = (x_ref[...] > 0.0).astype(jnp.float32)
    col = lax.broadcasted_iota(jnp.int32, (_IN_COLS, 128), 1) // CELLS_PER_COLUMN
    row = lax.broadcasted_iota(jnp.int32, (_IN_COLS, 128), 0)
    bmat = (col == row).astype(jnp.float32)
    o_ref[...] = jax.lax.dot_general(
        m, bmat, (((1,), (0,)), ((), ())),
        preferred_element_type=jnp.float32,
    )


_tc_expand = pl.pallas_call(
    _tc_expand_body,
    out_shape=jax.ShapeDtypeStruct((_ROWS, 128), jnp.float32),
    grid=(_ROWS // _BR,),
    in_specs=[pl.BlockSpec((_BR, _IN_COLS), lambda i: (i, 0))],
    out_specs=pl.BlockSpec((_BR, 128), lambda i: (i, 0)),
)


def _tc_zero_body(o_ref):
    o_ref[...] = jnp.zeros_like(o_ref)


_ZBLOCK = NUM_CELLS // 8

_tc_zeros = pl.pallas_call(
    _tc_zero_body,
    out_shape=jax.ShapeDtypeStruct((NUM_CELLS,), jnp.float32),
    grid=(NUM_CELLS // _ZBLOCK,),
    out_specs=pl.BlockSpec((_ZBLOCK,), lambda i: (i,)),
)


def kernel(active_columns):
    x2 = active_columns.reshape(_ROWS, _IN_COLS)
    new_active = _tc_expand(x2).reshape(NUM_CELLS)
    new_predictive = _tc_zeros()
    return (new_active, new_predictive)
